# Initial kernel scaffold; baseline (speedup 1.0000x reference)
import jax
import jax.numpy as jnp
from jax.experimental import pallas as pl
from jax.experimental.pallas import tpu as pltpu

_SCORE_THRESHOLD = 0.01
_NMS_THRESHOLD = 0.5
_MAX_DET = 100
_NEG = jnp.float32(-1e30)
_LANES = 128


def _stage1_body(cls_ref, s_ref, l_ref):
    cls = cls_ref[0]
    maxv = jnp.max(cls, axis=-1, keepdims=True)
    iota = jax.lax.broadcasted_iota(jnp.int32, cls.shape, 1)
    lbl = jnp.min(jnp.where(cls == maxv, iota, 2 ** 30), axis=-1, keepdims=True)
    s_ref[0] = jnp.where(maxv > _SCORE_THRESHOLD, maxv, _NEG)
    l_ref[0] = lbl


def _nms_body(s_ref, l_ref, p_ref, bo_ref, so_ref, lo_ref):
    rows = s_ref.shape[1]
    s0 = s_ref[0]
    lbls = l_ref[0]
    x1 = p_ref[0, 0]
    y1 = p_ref[0, 1]
    x2 = p_ref[0, 2]
    y2 = p_ref[0, 3]
    area2 = jnp.maximum(x2 - x1, 0.0) * jnp.maximum(y2 - y1, 0.0)
    flat = (jax.lax.broadcasted_iota(jnp.int32, (rows, _LANES), 0) * _LANES
            + jax.lax.broadcasted_iota(jnp.int32, (rows, _LANES), 1))
    lane = jax.lax.broadcasted_iota(jnp.int32, (1, _LANES), 1)

    def body(i, carry):
        s, sacc, lacc, bacc = carry
        m = jnp.max(s)
        keep = m > _NEG / 2.0
        idx = jnp.min(jnp.where(s == m, flat, 2 ** 30))
        isb = flat == idx
        bx1 = jnp.sum(jnp.where(isb, x1, 0.0))
        by1 = jnp.sum(jnp.where(isb, y1, 0.0))
        bx2 = jnp.sum(jnp.where(isb, x2, 0.0))
        by2 = jnp.sum(jnp.where(isb, y2, 0.0))
        lblv = jnp.sum(jnp.where(isb, lbls, 0))
        xx1 = jnp.maximum(bx1, x1)
        yy1 = jnp.maximum(by1, y1)
        xx2 = jnp.minimum(bx2, x2)
        yy2 = jnp.minimum(by2, y2)
        inter = jnp.maximum(xx2 - xx1, 0.0) * jnp.maximum(yy2 - yy1, 0.0)
        area1 = jnp.maximum(bx2 - bx1, 0.0) * jnp.maximum(by2 - by1, 0.0)
        union = area1 + area2 - inter
        iou = inter / jnp.maximum(union, 1e-8)
        supp = iou > _NMS_THRESHOLD
        s = jnp.where(supp | isb, _NEG, s)
        oh = lane == i
        sacc = sacc + jnp.where(oh, jnp.where(keep, m, -1.0), 0.0)
        lacc = lacc + jnp.where(oh, jnp.where(keep, lblv, -1), 0)
        coords = jnp.concatenate(
            [jnp.full((1, 1), v, jnp.float32) for v in (bx1, by1, bx2, by2)],
            axis=0)
        cvals = jnp.where(keep, coords, -1.0)
        bacc = bacc + jnp.where(oh, cvals, 0.0)
        return s, sacc, lacc, bacc

    init = (s0,
            jnp.zeros((1, _LANES), jnp.float32),
            jnp.zeros((1, _LANES), jnp.int32),
            jnp.zeros((4, _LANES), jnp.float32))
    _, sacc, lacc, bacc = jax.lax.fori_loop(0, _MAX_DET, body, init)
    so_ref[...] = sacc
    lo_ref[...] = lacc
    bo_ref[0] = bacc


def kernel(boxes, classification):
    B, N, C = classification.shape
    npad = pl.cdiv(N, _LANES) * _LANES
    rows = npad // _LANES

    scores, labels = pl.pallas_call(
        _stage1_body,
        grid=(B,),
        in_specs=[pl.BlockSpec((1, N, C), lambda b: (b, 0, 0))],
        out_specs=[pl.BlockSpec((1, N, 1), lambda b: (b, 0, 0)),
                   pl.BlockSpec((1, N, 1), lambda b: (b, 0, 0))],
        out_shape=[jax.ShapeDtypeStruct((B, N, 1), jnp.float32),
                   jax.ShapeDtypeStruct((B, N, 1), jnp.int32)],
    )(classification)

    scores = scores.reshape(B, N)
    labels = labels.reshape(B, N)
    sp = jnp.concatenate(
        [scores, jnp.full((B, npad - N), _NEG, jnp.float32)], axis=1
    ).reshape(B, rows, _LANES)
    lp = jnp.concatenate(
        [labels, jnp.zeros((B, npad - N), jnp.int32)], axis=1
    ).reshape(B, rows, _LANES)
    planes = jnp.moveaxis(boxes, 2, 1)
    pp = jnp.concatenate(
        [planes, jnp.zeros((B, 4, npad - N), jnp.float32)], axis=2
    ).reshape(B, 4, rows, _LANES)

    bo, so, lo = pl.pallas_call(
        _nms_body,
        grid=(B,),
        in_specs=[pl.BlockSpec((1, rows, _LANES), lambda b: (b, 0, 0)),
                  pl.BlockSpec((1, rows, _LANES), lambda b: (b, 0, 0)),
                  pl.BlockSpec((1, 4, rows, _LANES), lambda b: (b, 0, 0, 0))],
        out_specs=[pl.BlockSpec((1, 4, _LANES), lambda b: (b, 0, 0)),
                   pl.BlockSpec((1, _LANES), lambda b: (b, 0)),
                   pl.BlockSpec((1, _LANES), lambda b: (b, 0))],
        out_shape=[jax.ShapeDtypeStruct((B, 4, _LANES), jnp.float32),
                   jax.ShapeDtypeStruct((B, _LANES), jnp.float32),
                   jax.ShapeDtypeStruct((B, _LANES), jnp.int32)],
    )(sp, lp, pp)

    out_boxes = jnp.moveaxis(bo, 1, 2)[:, :_MAX_DET, :]
    out_scores = so[:, :_MAX_DET]
    out_labels = lo[:, :_MAX_DET]
    return out_boxes, out_scores, out_labels


# speedup vs baseline: 6.4371x; 6.4371x over previous
import jax
import jax.numpy as jnp
from jax.experimental import pallas as pl
from jax.experimental.pallas import tpu as pltpu

_SCORE_THRESHOLD = 0.01
_NMS_THRESHOLD = 0.5
_MAX_DET = 100
_NEG = -1e30
_LANES = 128


def _stage1_body(cls_ref, s_ref, l_ref):
    cls = cls_ref[0]
    maxv = jnp.max(cls, axis=-1, keepdims=True)
    iota = jax.lax.broadcasted_iota(jnp.int32, cls.shape, 1)
    lbl = jnp.min(jnp.where(cls == maxv, iota, 2 ** 30), axis=-1, keepdims=True)
    s_ref[0] = jnp.where(maxv > _SCORE_THRESHOLD, maxv, _NEG)
    l_ref[0] = lbl


def _nms_body(s_ref, l_ref, p_ref, bo_ref, so_ref, lo_ref):
    rows = s_ref.shape[1]
    s0 = s_ref[0]
    lbls = l_ref[0]
    x1 = p_ref[0, 0]
    y1 = p_ref[0, 1]
    x2 = p_ref[0, 2]
    y2 = p_ref[0, 3]
    area2 = jnp.maximum(x2 - x1, 0.0) * jnp.maximum(y2 - y1, 0.0)
    flat = (jax.lax.broadcasted_iota(jnp.int32, (rows, _LANES), 0) * _LANES
            + jax.lax.broadcasted_iota(jnp.int32, (rows, _LANES), 1))
    lane = jax.lax.broadcasted_iota(jnp.int32, (1, _LANES), 1)

    def body(i, carry):
        s, sacc, lacc, bacc = carry
        m = jnp.max(s)
        keep = m > _NEG / 2.0
        idx = jnp.min(jnp.where(s == m, flat, 2 ** 30))
        isb = flat == idx
        bx1 = jnp.sum(jnp.where(isb, x1, 0.0))
        by1 = jnp.sum(jnp.where(isb, y1, 0.0))
        bx2 = jnp.sum(jnp.where(isb, x2, 0.0))
        by2 = jnp.sum(jnp.where(isb, y2, 0.0))
        lblv = jnp.sum(jnp.where(isb, lbls, 0))
        xx1 = jnp.maximum(bx1, x1)
        yy1 = jnp.maximum(by1, y1)
        xx2 = jnp.minimum(bx2, x2)
        yy2 = jnp.minimum(by2, y2)
        inter = jnp.maximum(xx2 - xx1, 0.0) * jnp.maximum(yy2 - yy1, 0.0)
        area1 = jnp.maximum(bx2 - bx1, 0.0) * jnp.maximum(by2 - by1, 0.0)
        union = area1 + area2 - inter
        iou = inter / jnp.maximum(union, 1e-8)
        supp = iou > _NMS_THRESHOLD
        s = jnp.where(supp | isb, _NEG, s)
        oh = lane == i
        sacc = sacc + jnp.where(oh, jnp.where(keep, m, -1.0), 0.0)
        lacc = lacc + jnp.where(oh, jnp.where(keep, lblv, -1), 0)
        coords = jnp.concatenate(
            [jnp.full((1, 1), v, jnp.float32) for v in (bx1, by1, bx2, by2)],
            axis=0)
        cvals = jnp.where(keep, coords, -1.0)
        bacc = bacc + jnp.where(oh, cvals, 0.0)
        return s, sacc, lacc, bacc

    init = (s0,
            jnp.zeros((1, _LANES), jnp.float32),
            jnp.zeros((1, _LANES), jnp.int32),
            jnp.zeros((4, _LANES), jnp.float32))
    _, sacc, lacc, bacc = jax.lax.fori_loop(0, _MAX_DET, body, init)
    so_ref[0] = sacc
    lo_ref[0] = lacc
    bo_ref[0] = bacc


def kernel(boxes, classification):
    B, N, C = classification.shape
    npad = pl.cdiv(N, _LANES) * _LANES
    rows = npad // _LANES

    n_chunks = 10
    chunk = N // n_chunks
    scores, labels = pl.pallas_call(
        _stage1_body,
        grid=(B, n_chunks),
        in_specs=[pl.BlockSpec((1, chunk, C), lambda b, c: (b, c, 0))],
        out_specs=[pl.BlockSpec((1, chunk, 1), lambda b, c: (b, c, 0)),
                   pl.BlockSpec((1, chunk, 1), lambda b, c: (b, c, 0))],
        out_shape=[jax.ShapeDtypeStruct((B, N, 1), jnp.float32),
                   jax.ShapeDtypeStruct((B, N, 1), jnp.int32)],
    )(classification)

    scores = scores.reshape(B, N)
    labels = labels.reshape(B, N)
    sp = jnp.concatenate(
        [scores, jnp.full((B, npad - N), _NEG, jnp.float32)], axis=1
    ).reshape(B, rows, _LANES)
    lp = jnp.concatenate(
        [labels, jnp.zeros((B, npad - N), jnp.int32)], axis=1
    ).reshape(B, rows, _LANES)
    planes = jnp.moveaxis(boxes, 2, 1)
    pp = jnp.concatenate(
        [planes, jnp.zeros((B, 4, npad - N), jnp.float32)], axis=2
    ).reshape(B, 4, rows, _LANES)

    bo, so, lo = pl.pallas_call(
        _nms_body,
        grid=(B,),
        in_specs=[pl.BlockSpec((1, rows, _LANES), lambda b: (b, 0, 0)),
                  pl.BlockSpec((1, rows, _LANES), lambda b: (b, 0, 0)),
                  pl.BlockSpec((1, 4, rows, _LANES), lambda b: (b, 0, 0, 0))],
        out_specs=[pl.BlockSpec((1, 4, _LANES), lambda b: (b, 0, 0)),
                   pl.BlockSpec((1, 1, _LANES), lambda b: (b, 0, 0)),
                   pl.BlockSpec((1, 1, _LANES), lambda b: (b, 0, 0))],
        out_shape=[jax.ShapeDtypeStruct((B, 4, _LANES), jnp.float32),
                   jax.ShapeDtypeStruct((B, 1, _LANES), jnp.float32),
                   jax.ShapeDtypeStruct((B, 1, _LANES), jnp.int32)],
    )(sp, lp, pp)

    out_boxes = jnp.moveaxis(bo, 1, 2)[:, :_MAX_DET, :]
    out_scores = so[:, 0, :_MAX_DET]
    out_labels = lo[:, 0, :_MAX_DET]
    return out_boxes, out_scores, out_labels
